# baseline (speedup 1.0000x reference)
import functools

import jax
import jax.numpy as jnp
from jax import lax
from jax.experimental import pallas as pl
from jax.experimental.pallas import tpu as pltpu
from jax.experimental.pallas import tpu_sc as plsc

VOCAB = 1000
EMBED = 128
UNITS = 512
B = 64
T = 128
GATES = 4 * UNITS
CHUNK = 8
NCHUNK = T // CHUNK
MB = CHUNK * B


def _make_gather(V, D, N):
    info = plsc.get_sparse_core_info()
    NW = info.num_cores * info.num_subcores
    n_per_w = N // NW
    mesh = plsc.VectorSubcoreMesh(core_axis_name="c", subcore_axis_name="s")

    @functools.partial(
        pl.kernel,
        mesh=mesh,
        out_type=jax.ShapeDtypeStruct((N, D), jnp.float32),
        scratch_types=[
            pltpu.VMEM((n_per_w,), jnp.int32),
            pltpu.VMEM((n_per_w, D), jnp.float32),
            pltpu.SemaphoreType.DMA,
        ],
    )
    def gk(table_hbm, idx_hbm, out_hbm, idx_v, rows_v, sem):
        wid = lax.axis_index("s") * info.num_cores + lax.axis_index("c")
        base = wid * n_per_w
        pltpu.sync_copy(idx_hbm.at[pl.ds(base, n_per_w)], idx_v)
        pltpu.async_copy(table_hbm.at[idx_v], rows_v, sem).wait()
        pltpu.sync_copy(rows_v, out_hbm.at[pl.ds(base, n_per_w)])

    return gk


@functools.lru_cache(maxsize=None)
def _gather_fn():
    return _make_gather(VOCAB, EMBED, B * T)


def _scan_body(x_ref, wk_ref, rk_ref, bias_ref, wd_ref, bd_ref,
               hin_ref, cin_ref, out_ref, hout_ref, cout_ref,
               h_ref, c_ref, hs_ref):
    cidx = pl.program_id(0)

    @pl.when(cidx == 0)
    def _():
        h_ref[...] = hin_ref[...]
        c_ref[...] = cin_ref[...]

    xz = jnp.dot(
        x_ref[...], wk_ref[...], preferred_element_type=jnp.float32
    ) + bias_ref[...]

    c = c_ref[...]
    h = h_ref[...]
    for s in range(CHUNK):
        z = xz[s * B : (s + 1) * B, :] + jnp.dot(
            h.astype(jnp.bfloat16), rk_ref[...],
            preferred_element_type=jnp.float32,
        )
        i = 0.5 * jnp.tanh(0.5 * z[:, :UNITS]) + 0.5
        f = 0.5 * jnp.tanh(0.5 * z[:, UNITS : 2 * UNITS]) + 0.5
        g = jnp.tanh(z[:, 2 * UNITS : 3 * UNITS])
        o = 0.5 * jnp.tanh(0.5 * z[:, 3 * UNITS :]) + 0.5
        c = f * c + i * g
        h = o * jnp.tanh(c)
        hs_ref[s * B : (s + 1) * B, :] = h.astype(jnp.bfloat16)
    h_ref[...] = h
    c_ref[...] = c
    hout_ref[...] = h
    cout_ref[...] = c

    logits = jnp.dot(
        hs_ref[...], wd_ref[...], preferred_element_type=jnp.float32
    ) + bd_ref[...]
    out_ref[...] = logits.reshape(CHUNK, B, VOCAB)


NSEG = 2
SEG_CHUNKS = NCHUNK // NSEG
TSEG = T // NSEG


def _scan_seg(seg, x, kern, rkb, bias2, wdb, bd2, h_in, c_in):
    return pl.pallas_call(
        _scan_body,
        grid=(SEG_CHUNKS,),
        in_specs=[
            pl.BlockSpec((MB, EMBED), lambda c: (c + seg * SEG_CHUNKS, 0)),
            pl.BlockSpec((EMBED, GATES), lambda c: (0, 0)),
            pl.BlockSpec((UNITS, GATES), lambda c: (0, 0)),
            pl.BlockSpec((1, GATES), lambda c: (0, 0)),
            pl.BlockSpec((UNITS, VOCAB), lambda c: (0, 0)),
            pl.BlockSpec((1, VOCAB), lambda c: (0, 0)),
            pl.BlockSpec((B, UNITS), lambda c: (0, 0)),
            pl.BlockSpec((B, UNITS), lambda c: (0, 0)),
        ],
        out_specs=(
            pl.BlockSpec((CHUNK, B, VOCAB), lambda c: (c, 0, 0)),
            pl.BlockSpec((B, UNITS), lambda c: (0, 0)),
            pl.BlockSpec((B, UNITS), lambda c: (0, 0)),
        ),
        out_shape=(
            jax.ShapeDtypeStruct((TSEG, B, VOCAB), jnp.float32),
            jax.ShapeDtypeStruct((B, UNITS), jnp.float32),
            jax.ShapeDtypeStruct((B, UNITS), jnp.float32),
        ),
        scratch_shapes=[
            pltpu.VMEM((B, UNITS), jnp.float32),
            pltpu.VMEM((B, UNITS), jnp.float32),
            pltpu.VMEM((MB, UNITS), jnp.bfloat16),
        ],
    )(x, kern, rkb, bias2, wdb, bd2, h_in, c_in)


def kernel(inputs, W_emb, kernel, rec_kernel, bias, W_dense, b_dense):
    flat_idx = inputs.T.reshape(B * T)
    x = _gather_fn()(W_emb, flat_idx)
    rkb = rec_kernel.astype(jnp.bfloat16)
    wdb = W_dense.astype(jnp.bfloat16)
    bias2 = bias.reshape(1, GATES)
    bd2 = b_dense.reshape(1, VOCAB)
    h = jnp.zeros((B, UNITS), jnp.float32)
    c = jnp.zeros((B, UNITS), jnp.float32)
    parts = []
    for seg in range(NSEG):
        logits_seg, h, c = _scan_seg(
            seg, x, kernel, rkb, bias2, wdb, bd2, h, c
        )
        parts.append(jnp.swapaxes(logits_seg, 0, 1))
    return jnp.concatenate(parts, axis=1)


# speedup vs baseline: 1.2046x; 1.2046x over previous
import functools

import jax
import jax.numpy as jnp
from jax import lax
from jax.experimental import pallas as pl
from jax.experimental.pallas import tpu as pltpu
from jax.experimental.pallas import tpu_sc as plsc

VOCAB = 1000
EMBED = 128
UNITS = 512
B = 64
T = 128
GATES = 4 * UNITS
CHUNK = 8
NCHUNK = T // CHUNK
MB = CHUNK * B


def _make_gather(V, D, N):
    info = plsc.get_sparse_core_info()
    NW = info.num_cores * info.num_subcores
    n_per_w = N // NW
    mesh = plsc.VectorSubcoreMesh(core_axis_name="c", subcore_axis_name="s")

    @functools.partial(
        pl.kernel,
        mesh=mesh,
        out_type=jax.ShapeDtypeStruct((N, D), jnp.float32),
        scratch_types=[
            pltpu.VMEM((n_per_w,), jnp.int32),
            pltpu.VMEM((n_per_w, D), jnp.float32),
            pltpu.SemaphoreType.DMA,
        ],
    )
    def gk(table_hbm, idx_hbm, out_hbm, idx_v, rows_v, sem):
        wid = lax.axis_index("s") * info.num_cores + lax.axis_index("c")
        base = wid * n_per_w
        pltpu.sync_copy(idx_hbm.at[pl.ds(base, n_per_w)], idx_v)
        pltpu.async_copy(table_hbm.at[idx_v], rows_v, sem).wait()
        pltpu.sync_copy(rows_v, out_hbm.at[pl.ds(base, n_per_w)])

    return gk


@functools.lru_cache(maxsize=None)
def _gather_fn():
    return _make_gather(VOCAB, EMBED, B * T)


def _scan_body(x_ref, wk_ref, rk_ref, bias_ref, wd_ref, bd_ref, out_ref,
               h_ref, c_ref, hs_ref):
    cidx = pl.program_id(0)

    @pl.when(cidx == 0)
    def _():
        h_ref[...] = jnp.zeros_like(h_ref)
        c_ref[...] = jnp.zeros_like(c_ref)

    xz = jnp.dot(
        x_ref[...], wk_ref[...], preferred_element_type=jnp.float32
    ) + bias_ref[...]

    c = c_ref[...]
    h = h_ref[...]
    for s in range(CHUNK):
        z = xz[s * B : (s + 1) * B, :] + jnp.dot(
            h.astype(jnp.bfloat16), rk_ref[...],
            preferred_element_type=jnp.float32,
        )
        i = 0.5 * jnp.tanh(0.5 * z[:, :UNITS]) + 0.5
        f = 0.5 * jnp.tanh(0.5 * z[:, UNITS : 2 * UNITS]) + 0.5
        g = jnp.tanh(z[:, 2 * UNITS : 3 * UNITS])
        o = 0.5 * jnp.tanh(0.5 * z[:, 3 * UNITS :]) + 0.5
        c = f * c + i * g
        h = o * jnp.tanh(c)
        hs_ref[s * B : (s + 1) * B, :] = h.astype(jnp.bfloat16)
    h_ref[...] = h
    c_ref[...] = c

    logits = jnp.dot(
        hs_ref[...], wd_ref[...], preferred_element_type=jnp.float32
    ) + bd_ref[...]
    out_ref[...] = logits.reshape(CHUNK, B, VOCAB)


def _scan(x, kern, rec_kernel, bias, W_dense, b_dense):
    return pl.pallas_call(
        _scan_body,
        grid=(NCHUNK,),
        in_specs=[
            pl.BlockSpec((MB, EMBED), lambda c: (c, 0)),
            pl.BlockSpec((EMBED, GATES), lambda c: (0, 0)),
            pl.BlockSpec((UNITS, GATES), lambda c: (0, 0)),
            pl.BlockSpec((1, GATES), lambda c: (0, 0)),
            pl.BlockSpec((UNITS, VOCAB), lambda c: (0, 0)),
            pl.BlockSpec((1, VOCAB), lambda c: (0, 0)),
        ],
        out_specs=pl.BlockSpec((CHUNK, B, VOCAB), lambda c: (c, 0, 0)),
        out_shape=jax.ShapeDtypeStruct((T, B, VOCAB), jnp.float32),
        scratch_shapes=[
            pltpu.VMEM((B, UNITS), jnp.float32),
            pltpu.VMEM((B, UNITS), jnp.float32),
            pltpu.VMEM((MB, UNITS), jnp.bfloat16),
        ],
    )(
        x,
        kern,
        rec_kernel.astype(jnp.bfloat16),
        bias.reshape(1, GATES),
        W_dense.astype(jnp.bfloat16),
        b_dense.reshape(1, VOCAB),
    )


def kernel(inputs, W_emb, kernel, rec_kernel, bias, W_dense, b_dense):
    flat_idx = inputs.T.reshape(B * T)
    x = _gather_fn()(W_emb, flat_idx)
    logits_tbv = _scan(x, kernel, rec_kernel, bias, W_dense, b_dense)
    return jnp.swapaxes(logits_tbv, 0, 1)


# speedup vs baseline: 1.2188x; 1.0118x over previous
import functools

import jax
import jax.numpy as jnp
from jax import lax
from jax.experimental import pallas as pl
from jax.experimental.pallas import tpu as pltpu
from jax.experimental.pallas import tpu_sc as plsc

VOCAB = 1000
EMBED = 128
UNITS = 512
B = 64
T = 128
GATES = 4 * UNITS
CHUNK = 16
NCHUNK = T // CHUNK
MB = CHUNK * B


def _make_gather(V, D, N):
    info = plsc.get_sparse_core_info()
    NW = info.num_cores * info.num_subcores
    n_per_w = N // NW
    mesh = plsc.VectorSubcoreMesh(core_axis_name="c", subcore_axis_name="s")

    @functools.partial(
        pl.kernel,
        mesh=mesh,
        out_type=jax.ShapeDtypeStruct((N, D), jnp.float32),
        scratch_types=[
            pltpu.VMEM((n_per_w,), jnp.int32),
            pltpu.VMEM((n_per_w, D), jnp.float32),
            pltpu.SemaphoreType.DMA,
        ],
    )
    def gk(table_hbm, idx_hbm, out_hbm, idx_v, rows_v, sem):
        wid = lax.axis_index("s") * info.num_cores + lax.axis_index("c")
        base = wid * n_per_w
        pltpu.sync_copy(idx_hbm.at[pl.ds(base, n_per_w)], idx_v)
        pltpu.async_copy(table_hbm.at[idx_v], rows_v, sem).wait()
        pltpu.sync_copy(rows_v, out_hbm.at[pl.ds(base, n_per_w)])

    return gk


@functools.lru_cache(maxsize=None)
def _gather_fn():
    return _make_gather(VOCAB, EMBED, B * T)


def _scan_body(x_ref, wk_ref, rk_ref, bias_ref, wd_ref, bd_ref, out_ref,
               h_ref, c_ref, hs_ref):
    cidx = pl.program_id(0)

    @pl.when(cidx == 0)
    def _():
        h_ref[...] = jnp.zeros_like(h_ref)
        c_ref[...] = jnp.zeros_like(c_ref)

    xz = jnp.dot(
        x_ref[...], wk_ref[...], preferred_element_type=jnp.float32
    ) + bias_ref[...]

    c = c_ref[...]
    h = h_ref[...]
    for s in range(CHUNK):
        z = xz[s * B : (s + 1) * B, :] + jnp.dot(
            h.astype(jnp.bfloat16), rk_ref[...],
            preferred_element_type=jnp.float32,
        )
        i = 0.5 * jnp.tanh(0.5 * z[:, :UNITS]) + 0.5
        f = 0.5 * jnp.tanh(0.5 * z[:, UNITS : 2 * UNITS]) + 0.5
        g = jnp.tanh(z[:, 2 * UNITS : 3 * UNITS])
        o = 0.5 * jnp.tanh(0.5 * z[:, 3 * UNITS :]) + 0.5
        c = f * c + i * g
        h = o * jnp.tanh(c)
        hs_ref[s * B : (s + 1) * B, :] = h.astype(jnp.bfloat16)
    h_ref[...] = h
    c_ref[...] = c

    logits = jnp.dot(
        hs_ref[...], wd_ref[...], preferred_element_type=jnp.float32
    ) + bd_ref[...]
    out_ref[...] = logits.reshape(CHUNK, B, VOCAB)


def _scan(x, kern, rec_kernel, bias, W_dense, b_dense):
    return pl.pallas_call(
        _scan_body,
        grid=(NCHUNK,),
        in_specs=[
            pl.BlockSpec((MB, EMBED), lambda c: (c, 0)),
            pl.BlockSpec((EMBED, GATES), lambda c: (0, 0)),
            pl.BlockSpec((UNITS, GATES), lambda c: (0, 0)),
            pl.BlockSpec((1, GATES), lambda c: (0, 0)),
            pl.BlockSpec((UNITS, VOCAB), lambda c: (0, 0)),
            pl.BlockSpec((1, VOCAB), lambda c: (0, 0)),
        ],
        out_specs=pl.BlockSpec((CHUNK, B, VOCAB), lambda c: (c, 0, 0)),
        out_shape=jax.ShapeDtypeStruct((T, B, VOCAB), jnp.float32),
        scratch_shapes=[
            pltpu.VMEM((B, UNITS), jnp.float32),
            pltpu.VMEM((B, UNITS), jnp.float32),
            pltpu.VMEM((MB, UNITS), jnp.bfloat16),
        ],
    )(
        x,
        kern,
        rec_kernel.astype(jnp.bfloat16),
        bias.reshape(1, GATES),
        W_dense.astype(jnp.bfloat16),
        b_dense.reshape(1, VOCAB),
    )


def kernel(inputs, W_emb, kernel, rec_kernel, bias, W_dense, b_dense):
    flat_idx = inputs.T.reshape(B * T)
    x = _gather_fn()(W_emb, flat_idx)
    logits_tbv = _scan(x, kernel, rec_kernel, bias, W_dense, b_dense)
    return jnp.swapaxes(logits_tbv, 0, 1)
